# baseline (speedup 1.0000x reference)
import functools

import jax
import jax.numpy as jnp
from jax import lax
from jax.experimental import pallas as pl
from jax.experimental.pallas import tpu as pltpu
from jax.experimental.pallas import tpu_sc as plsc

_NUM_EMB = 1024
_DIM = 64
_ROWS = 1024
_BPG = 2
_COMMIT = 0.25
_SCW = 128


def _vq_tc_body(xt_ref, wt_ref, idx_ref, loss_ref):
    i = pl.program_id(0)
    nb = pl.num_programs(0)
    wt = wt_ref[...]
    wsq = wt * wt
    ones = jnp.ones((_DIM, 1), jnp.float32)
    w2 = lax.dot_general(wsq, ones, (((0,), (0,)), ((), ())),
                         preferred_element_type=jnp.float32)
    wt2 = wt + wt
    part = jnp.float32(0.0)
    for s in range(_BPG):
        xt = xt_ref[s]
        x2 = jnp.sum(xt * xt, axis=0, keepdims=True)
        mm2 = lax.dot_general(wt2, xt, (((0,), (0,)), ((), ())),
                              preferred_element_type=jnp.float32)
        d = (x2 + w2) - mm2
        mn = jnp.min(d, axis=0, keepdims=True)
        codes = lax.broadcasted_iota(jnp.int32, d.shape, 0)
        idx_ref[s] = jnp.min(jnp.where(d == mn, codes, _NUM_EMB),
                             axis=0, keepdims=True)
        part = part + jnp.sum(mn)

    @pl.when(i == 0)
    def _():
        loss_ref[0, 0] = part

    @pl.when(i > 0)
    def _():
        loss_ref[0, 0] += part


def _tc_argmin(xt, wt, start, num):
    nb = num // _BPG
    off = start // _BPG
    return pl.pallas_call(
        _vq_tc_body,
        grid=(nb,),
        in_specs=[
            pl.BlockSpec((_BPG, _DIM, _ROWS), lambda i: (i + off, 0, 0)),
            pl.BlockSpec((_DIM, _NUM_EMB), lambda i: (0, 0)),
        ],
        out_specs=[
            pl.BlockSpec((_BPG, 1, _ROWS), lambda i: (i, 0, 0)),
            pl.BlockSpec(memory_space=pltpu.SMEM),
        ],
        out_shape=[
            jax.ShapeDtypeStruct((num, 1, _ROWS), jnp.int32),
            jax.ShapeDtypeStruct((1, 1), jnp.float32),
        ],
    )(xt, wt)


def _sc_gather(wpad, idx2d):
    info = plsc.get_sparse_core_info()
    nc, ns = info.num_cores, info.num_subcores
    nw = nc * ns
    m = idx2d.shape[0] * idx2d.shape[1]
    bpw = m // nw
    ch = idx2d.shape[1]
    nch = bpw // ch

    mesh = plsc.VectorSubcoreMesh(core_axis_name="c", subcore_axis_name="s")

    @functools.partial(
        pl.kernel, mesh=mesh,
        out_type=jax.ShapeDtypeStruct((m, _SCW), jnp.float32),
        scratch_types=[
            pltpu.VMEM((nch, ch), jnp.int32),
            pltpu.VMEM((bpw, _SCW), jnp.float32),
            pltpu.SemaphoreType.DMA,
        ],
    )
    def gather_k(idx_hbm, table_hbm, out_hbm, idx_v, rows_v, sem):
        wid = lax.axis_index("s") * nc + lax.axis_index("c")
        pltpu.sync_copy(idx_hbm.at[pl.ds(wid * nch, nch)], idx_v)
        copies = [pltpu.async_copy(table_hbm.at[idx_v.at[j]],
                                   rows_v.at[pl.ds(j * ch, ch)], sem)
                  for j in range(nch)]
        for c in copies:
            c.wait()
        pltpu.sync_copy(rows_v, out_hbm.at[pl.ds(wid * bpw, bpw)])

    return gather_k(idx2d, wpad)


def kernel(inputs, weight):
    shp = inputs.shape
    xt = inputs.transpose(0, 2, 1)
    wt = weight.T
    wpad = jnp.pad(weight, ((0, 0), (0, _SCW - _DIM)))
    half = shp[0] // 2
    mh = half * shp[1]
    idx_a, sum_a = _tc_argmin(xt, wt, 0, half)
    quant_a = _sc_gather(wpad, idx_a.reshape(mh // _SCW, _SCW))
    idx_b, sum_b = _tc_argmin(xt, wt, half, half)
    quant_b = _sc_gather(wpad, idx_b.reshape(mh // _SCW, _SCW))
    loss = ((sum_a[0, 0] + sum_b[0, 0])
            * ((1.0 + _COMMIT) / (shp[0] * shp[1] * _DIM)))
    idx = jnp.concatenate([idx_a, idx_b]).reshape(2 * mh, 1)
    quant = jnp.concatenate([quant_a, quant_b])[:, :_DIM]
    return idx, quant.reshape(shp), loss


# speedup vs baseline: 1.2291x; 1.2291x over previous
import functools

import jax
import jax.numpy as jnp
from jax import lax
from jax.experimental import pallas as pl
from jax.experimental.pallas import tpu as pltpu
from jax.experimental.pallas import tpu_sc as plsc

_NUM_EMB = 1024
_DIM = 64
_ROWS = 1024
_BPG = 2
_COMMIT = 0.25
_SCW = 128


def _vq_tc_body(xt_ref, wt_ref, idx_ref, loss_ref):
    i = pl.program_id(0)
    nb = pl.num_programs(0)
    wt = wt_ref[...]
    wsq = wt * wt
    ones = jnp.ones((_DIM, 1), jnp.float32)
    w2 = lax.dot_general(wsq, ones, (((0,), (0,)), ((), ())),
                         preferred_element_type=jnp.float32)
    wt2 = wt + wt
    part = jnp.float32(0.0)
    for s in range(_BPG):
        xt = xt_ref[s]
        x2 = jnp.sum(xt * xt, axis=0, keepdims=True)
        mm2 = lax.dot_general(wt2, xt, (((0,), (0,)), ((), ())),
                              preferred_element_type=jnp.float32)
        g = 8
        nstrips = _NUM_EMB // g
        minv = (x2 + lax.slice(w2, (0, 0), (g, 1))) \
            - lax.slice(mm2, (0, 0), (g, _ROWS))
        rowi = jnp.zeros((g, _ROWS), jnp.int32)
        for r in range(1, nstrips):
            dr = (x2 + lax.slice(w2, (r * g, 0), ((r + 1) * g, 1))) \
                - lax.slice(mm2, (r * g, 0), ((r + 1) * g, _ROWS))
            lt = dr < minv
            minv = jnp.minimum(minv, dr)
            rowi = jnp.where(lt, r, rowi)
        mn = jnp.min(minv, axis=0, keepdims=True)
        sub = lax.broadcasted_iota(jnp.int32, (g, _ROWS), 0)
        codes = rowi * g + sub
        idx_ref[s] = jnp.min(jnp.where(minv == mn, codes, _NUM_EMB),
                             axis=0, keepdims=True)
        part = part + jnp.sum(mn)

    @pl.when(i == 0)
    def _():
        loss_ref[0, 0] = part

    @pl.when(i > 0)
    def _():
        loss_ref[0, 0] += part

    @pl.when(i == nb - 1)
    def _():
        total = jnp.float32(_BPG * _ROWS * nb * _DIM)
        loss_ref[0, 0] = loss_ref[0, 0] * ((1.0 + _COMMIT) / total)


def _tc_argmin(xt, wt):
    nb = xt.shape[0] // _BPG
    return pl.pallas_call(
        _vq_tc_body,
        grid=(nb,),
        in_specs=[
            pl.BlockSpec((_BPG, _DIM, _ROWS), lambda i: (i, 0, 0)),
            pl.BlockSpec((_DIM, _NUM_EMB), lambda i: (0, 0)),
        ],
        out_specs=[
            pl.BlockSpec((_BPG, 1, _ROWS), lambda i: (i, 0, 0)),
            pl.BlockSpec(memory_space=pltpu.SMEM),
        ],
        out_shape=[
            jax.ShapeDtypeStruct((xt.shape[0], 1, _ROWS), jnp.int32),
            jax.ShapeDtypeStruct((1, 1), jnp.float32),
        ],
    )(xt, wt)


def _sc_gather(wpad, idx2d):
    info = plsc.get_sparse_core_info()
    nc, ns = info.num_cores, info.num_subcores
    nw = nc * ns
    m = idx2d.shape[0] * idx2d.shape[1]
    bpw = m // nw
    ch = idx2d.shape[1]
    nch = bpw // ch

    mesh = plsc.VectorSubcoreMesh(core_axis_name="c", subcore_axis_name="s")

    @functools.partial(
        pl.kernel, mesh=mesh,
        out_type=jax.ShapeDtypeStruct((m, _SCW), jnp.float32),
        scratch_types=[
            pltpu.VMEM((nch, ch), jnp.int32),
            pltpu.VMEM((bpw, _SCW), jnp.float32),
            pltpu.SemaphoreType.DMA,
        ],
    )
    def gather_k(idx_hbm, table_hbm, out_hbm, idx_v, rows_v, sem):
        wid = lax.axis_index("s") * nc + lax.axis_index("c")
        pltpu.sync_copy(idx_hbm.at[pl.ds(wid * nch, nch)], idx_v)
        copies = [pltpu.async_copy(table_hbm.at[idx_v.at[j]],
                                   rows_v.at[pl.ds(j * ch, ch)], sem)
                  for j in range(nch)]
        for c in copies:
            c.wait()
        pltpu.sync_copy(rows_v, out_hbm.at[pl.ds(wid * bpw, bpw)])

    return gather_k(idx2d, wpad)


def kernel(inputs, weight):
    shp = inputs.shape
    xt = inputs.transpose(0, 2, 1)
    wt = weight.T
    idx3, loss = _tc_argmin(xt, wt)
    m = shp[0] * shp[1]
    wpad = jnp.pad(weight, ((0, 0), (0, _SCW - _DIM)))
    quant = _sc_gather(wpad, idx3.reshape(m // _SCW, _SCW))[:, :_DIM]
    return idx3.reshape(m, 1), quant.reshape(shp), loss[0, 0]


# speedup vs baseline: 1.2409x; 1.0096x over previous
import functools

import jax
import jax.numpy as jnp
from jax import lax
from jax.experimental import pallas as pl
from jax.experimental.pallas import tpu as pltpu
from jax.experimental.pallas import tpu_sc as plsc

_NUM_EMB = 1024
_DIM = 64
_ROWS = 1024
_BPG = 4
_COMMIT = 0.25
_SCW = 128


def _vq_tc_body(xt_ref, wt_ref, idx_ref, loss_ref):
    i = pl.program_id(0)
    nb = pl.num_programs(0)
    wt = wt_ref[...]
    wsq = wt * wt
    ones = jnp.ones((_DIM, 1), jnp.float32)
    w2 = lax.dot_general(wsq, ones, (((0,), (0,)), ((), ())),
                         preferred_element_type=jnp.float32)
    wt2 = wt + wt
    part = jnp.float32(0.0)
    for s in range(_BPG):
        xt = xt_ref[s]
        x2 = jnp.sum(xt * xt, axis=0, keepdims=True)
        mm2 = lax.dot_general(wt2, xt, (((0,), (0,)), ((), ())),
                              preferred_element_type=jnp.float32)
        g = 8
        nstrips = _NUM_EMB // g
        minv = (x2 + lax.slice(w2, (0, 0), (g, 1))) \
            - lax.slice(mm2, (0, 0), (g, _ROWS))
        rowi = jnp.zeros((g, _ROWS), jnp.int32)
        for r in range(1, nstrips):
            dr = (x2 + lax.slice(w2, (r * g, 0), ((r + 1) * g, 1))) \
                - lax.slice(mm2, (r * g, 0), ((r + 1) * g, _ROWS))
            lt = dr < minv
            minv = jnp.minimum(minv, dr)
            rowi = jnp.where(lt, r, rowi)
        mn = jnp.min(minv, axis=0, keepdims=True)
        sub = lax.broadcasted_iota(jnp.int32, (g, _ROWS), 0)
        codes = rowi * g + sub
        idx_ref[s] = jnp.min(jnp.where(minv == mn, codes, _NUM_EMB),
                             axis=0, keepdims=True)
        part = part + jnp.sum(mn)

    @pl.when(i == 0)
    def _():
        loss_ref[0, 0] = part

    @pl.when(i > 0)
    def _():
        loss_ref[0, 0] += part

    @pl.when(i == nb - 1)
    def _():
        total = jnp.float32(_BPG * _ROWS * nb * _DIM)
        loss_ref[0, 0] = loss_ref[0, 0] * ((1.0 + _COMMIT) / total)


def _tc_argmin(xt, wt):
    nb = xt.shape[0] // _BPG
    return pl.pallas_call(
        _vq_tc_body,
        grid=(nb,),
        in_specs=[
            pl.BlockSpec((_BPG, _DIM, _ROWS), lambda i: (i, 0, 0)),
            pl.BlockSpec((_DIM, _NUM_EMB), lambda i: (0, 0)),
        ],
        out_specs=[
            pl.BlockSpec((_BPG, 1, _ROWS), lambda i: (i, 0, 0)),
            pl.BlockSpec(memory_space=pltpu.SMEM),
        ],
        out_shape=[
            jax.ShapeDtypeStruct((xt.shape[0], 1, _ROWS), jnp.int32),
            jax.ShapeDtypeStruct((1, 1), jnp.float32),
        ],
    )(xt, wt)


def _sc_gather(wpad, idx2d):
    info = plsc.get_sparse_core_info()
    nc, ns = info.num_cores, info.num_subcores
    nw = nc * ns
    m = idx2d.shape[0] * idx2d.shape[1]
    bpw = m // nw
    ch = idx2d.shape[1]
    nch = bpw // ch

    mesh = plsc.VectorSubcoreMesh(core_axis_name="c", subcore_axis_name="s")

    @functools.partial(
        pl.kernel, mesh=mesh,
        out_type=jax.ShapeDtypeStruct((m, _SCW), jnp.float32),
        scratch_types=[
            pltpu.VMEM((nch, ch), jnp.int32),
            pltpu.VMEM((bpw, _SCW), jnp.float32),
            pltpu.SemaphoreType.DMA,
        ],
    )
    def gather_k(idx_hbm, table_hbm, out_hbm, idx_v, rows_v, sem):
        wid = lax.axis_index("s") * nc + lax.axis_index("c")
        pltpu.sync_copy(idx_hbm.at[pl.ds(wid * nch, nch)], idx_v)
        copies = [pltpu.async_copy(table_hbm.at[idx_v.at[j]],
                                   rows_v.at[pl.ds(j * ch, ch)], sem)
                  for j in range(nch)]
        for c in copies:
            c.wait()
        pltpu.sync_copy(rows_v, out_hbm.at[pl.ds(wid * bpw, bpw)])

    return gather_k(idx2d, wpad)


def kernel(inputs, weight):
    shp = inputs.shape
    xt = inputs.transpose(0, 2, 1)
    wt = weight.T
    idx3, loss = _tc_argmin(xt, wt)
    m = shp[0] * shp[1]
    wpad = jnp.pad(weight, ((0, 0), (0, _SCW - _DIM)))
    quant = _sc_gather(wpad, idx3.reshape(m // _SCW, _SCW))[:, :_DIM]
    return idx3.reshape(m, 1), quant.reshape(shp), loss[0, 0]
